# Initial kernel scaffold; baseline (speedup 1.0000x reference)
import jax
import jax.numpy as jnp
from jax.experimental import pallas as pl


def kernel(x, edge_index, edge_attr, batch_idx, global_features, node_W, node_b, edge_W, edge_b, conv_W, conv_att, conv_bias, ga_W0, ga_b0, ga_W1, ga_b1, ga_W2, ga_b2, out_W0, out_b0, out_W1, out_b1):
    raise NotImplementedError("write your pallas kernel here")


# speedup vs baseline: 1.0410x; 1.0410x over previous
import jax, jax.numpy as jnp
import numpy as np
HID, HEADS, LAYERS = 64, 4, 5
BN_SCALE = 1.0 / np.sqrt(1.0 + 1e-3)

def kernel(x, edge_index, edge_attr, batch_idx, global_features, node_W, node_b, edge_W, edge_b, conv_W, conv_att, conv_bias, ga_W0, ga_b0, ga_W1, ga_b1, ga_W2, ga_b2, out_W0, out_b0, out_W1, out_b1):
    row = edge_index[0]
    col = edge_index[1]
    E = row.shape[0]
    N = x.shape[0]
    Q = E // 4
    h = jax.nn.leaky_relu(x @ node_W + node_b, negative_slope=0.2)
    ea = jax.nn.leaky_relu(edge_attr @ edge_W + edge_b, negative_slope=0.2)
    for l in range(LAYERS):
        W = conv_W[l]
        att = conv_att[l]
        Wt, Wb = W[:HID], W[HID:]
        att_i = att[0, :, :HID]
        att_j = att[0, :, HID:]
        Ai = jnp.einsum('khd,hd->kh', Wt.reshape(HID, HEADS, HID), att_i)
        Aj = jnp.einsum('khd,hd->kh', Wt.reshape(HID, HEADS, HID), att_j)
        Bij = jnp.einsum('khd,hd->kh', Wb.reshape(HID, HEADS, HID), att_i + att_j)
        a_i = h @ Ai
        a_j = h @ Aj
        b_e = ea @ Bij
        alpha_raw = a_i[row] + a_j[col] + b_e
        alpha = jnp.exp(BN_SCALE * jax.nn.leaky_relu(alpha_raw, negative_slope=0.2))
        norm = jnp.zeros((N, HEADS), h.dtype).at[row].add(alpha)
        anorm = alpha / norm[row]
        Hn = h @ Wt
        EB = ea @ Wb
        XJ = Hn[col] + EB
        msg = XJ.reshape(E, HEADS, HID) * anorm[..., None]
        S = msg.reshape(Q, 4, HEADS, HID).sum(axis=1)
        Sp = jnp.transpose(S, (1, 0, 2)).reshape(E, HID)
        acc = jnp.zeros((N, HID), h.dtype).at[row].add(Sp)
        h = h + acc / 4.0 + conv_bias[l]
    g = global_features[batch_idx]
    feats = jnp.concatenate([h, g], axis=-1)
    a = jax.nn.relu((feats @ ga_W0 + ga_b0) * BN_SCALE)
    a = jax.nn.relu((a @ ga_W1 + ga_b1) * BN_SCALE)
    attn = jnp.exp(a @ ga_W2 + ga_b2)
    G = global_features.shape[0]
    norm = jnp.zeros((G, 1), h.dtype).at[batch_idx].add(attn)
    attn = attn / norm[batch_idx]
    h = h * attn
    pooled = jnp.zeros((G, HID), h.dtype).at[batch_idx].add(h)
    o = jax.nn.relu(pooled @ out_W0 + out_b0)
    o = o @ out_W1 + out_b1
    return jnp.squeeze(o, -1)


# speedup vs baseline: 4.0958x; 3.9345x over previous
import functools

import jax
import jax.numpy as jnp
import numpy as np
from jax import lax
from jax.experimental import pallas as pl
from jax.experimental.pallas import tpu as pltpu
from jax.experimental.pallas import tpu_sc as plsc

HID = 64
HEADS = 4
LAYERS = 5
BN_SCALE = np.float32(1.0 / np.sqrt(1.0 + 1e-3))

NC = 2
NS = 16
NW = NC * NS

CH = 128


def _lrelu(t):
    return jnp.maximum(t, t * np.float32(0.2))


def _p0_body(eattr, ew, eb, bij, ea_o, *be_o):
    t = jnp.dot(eattr[...], ew[...], preferred_element_type=jnp.float32)
    t = _lrelu(t + eb[...])
    ea_o[...] = t
    ball = jnp.dot(t, bij[...], preferred_element_type=jnp.float32)
    for l in range(LAYERS):
        be_o[l][...] = ball[:, 4 * l:4 * l + 4]


def _p1_body(x, nw, nb, h_o):
    t = jnp.dot(x[...], nw[...], preferred_element_type=jnp.float32)
    h_o[...] = _lrelu(t + nb[...])


def _t1a_body(h, aiaj, av_o):
    av = jnp.dot(h[...], aiaj[...], preferred_element_type=jnp.float32)
    av_o[...] = jnp.concatenate([av, jnp.zeros_like(av)], axis=1)


def _t1_body(h, acc2, bias, aiaj, h_o, av_o):
    hn = h[...] + (acc2[0] + acc2[1]) * np.float32(0.25) + bias[...]
    h_o[...] = hn
    av = jnp.dot(hn, aiaj[...], preferred_element_type=jnp.float32)
    av_o[...] = jnp.concatenate([av, jnp.zeros_like(av)], axis=1)


def _t2_body(hcol, ea, anorm, w, s_o):
    xj = jnp.dot(jnp.concatenate([hcol[...], ea[...]], axis=1), w[...],
                 preferred_element_type=jnp.float32)
    bq = s_o.shape[1]
    for h in range(HEADS):
        m = xj[:, HID * h:HID * h + HID] * anorm[:, h:h + 1]
        s_o[h, :, :] = (m[0:bq] + m[bq:2 * bq] + m[2 * bq:3 * bq]
                        + m[3 * bq:4 * bq])


def _make_tail1_body(n_real):
    def body(h, acc2, bias, bidx, gf, w0h, w0g, b0, w1, b1, w2, b2,
             h5_o, attn_o, normg_o):
        i = pl.program_id(0)
        bn = h.shape[0]
        h5 = h[...] + (acc2[0] + acc2[1]) * np.float32(0.25) + bias[...]
        h5_o[...] = h5
        bi = bidx[0, 0, :]
        oh = (bi[:, None] == lax.broadcasted_iota(
            jnp.int32, (bn, gf.shape[0]), 1)).astype(jnp.float32)
        gs = jnp.dot(gf[...], w0g[...], preferred_element_type=jnp.float32)
        a0 = jnp.dot(h5, w0h[...], preferred_element_type=jnp.float32) \
            + jnp.dot(oh, gs, preferred_element_type=jnp.float32) + b0[...]
        a0 = jnp.maximum(a0 * BN_SCALE, 0.0)
        a1 = jnp.dot(a0, w1[...], preferred_element_type=jnp.float32) + b1[...]
        a1 = jnp.maximum(a1 * BN_SCALE, 0.0)
        lg = jnp.dot(a1, w2[...], preferred_element_type=jnp.float32) + b2[...]
        at = jnp.exp(lg)
        grow = i * bn + lax.broadcasted_iota(jnp.int32, (bn, 1), 0)
        at = jnp.where(grow < n_real, at, 0.0)
        attn_o[...] = at

        @pl.when(i == 0)
        def _():
            normg_o[...] = jnp.zeros_like(normg_o)
        normg_o[...] += lax.dot_general(oh, at, (((0,), (0,)), ((), ())),
                                        preferred_element_type=jnp.float32)
    return body


def _tail2_body(h5, attn, normg, bidx, ow0, ob0, ow1, ob1, pooled_o, o_o):
    i = pl.program_id(0)
    bn = h5.shape[0]
    bi = bidx[0, 0, :]
    oh = (bi[:, None] == lax.broadcasted_iota(
        jnp.int32, (bn, normg.shape[0]), 1)).astype(jnp.float32)
    nrm = jnp.dot(oh, normg[...], preferred_element_type=jnp.float32)
    atn = attn[...] / nrm
    hw = h5[...] * atn

    @pl.when(i == 0)
    def _():
        pooled_o[...] = jnp.zeros_like(pooled_o)
    pooled_o[...] += lax.dot_general(oh, hw, (((0,), (0,)), ((), ())),
                                     preferred_element_type=jnp.float32)

    @pl.when(i == pl.num_programs(0) - 1)
    def _():
        p = pooled_o[...]
        t0 = jnp.maximum(
            jnp.dot(p, ow0[...], preferred_element_type=jnp.float32) + ob0[...],
            0.0)
        o_o[...] = jnp.dot(t0, ow1[...],
                           preferred_element_type=jnp.float32) + ob1[...]


def _full(shape):
    return pl.BlockSpec(shape, lambda i: (0,) * len(shape))


_GDN = lax.GatherDimensionNumbers(offset_dims=(), collapsed_slice_dims=(0,),
                                  start_index_map=(0,))


def _vgather(v, idx):
    return lax.gather(v, idx[:, None], _GDN, (1,),
                      mode=lax.GatherScatterMode.PROMISE_IN_BOUNDS)


def _sc_mesh():
    return plsc.VectorSubcoreMesh(core_axis_name="c", subcore_axis_name="s")


def _build_s1(E, NP, NFLATP):
    EPW = E // NW
    NCHUNK = EPW // CH
    NPAIR = NCHUNK // 2
    TAIL = EPW - NCHUNK * CH
    RED = NFLATP // NS

    def body(av16_h, h_h, row_h, col_h, be_h,
             alpha_o, hcol_o, norm2_o,
             idxrA, idxcA, idxrB, idxcB,
             avrA, avcA, avrB, avcB, hrowA, hrowB,
             bevA, bevB, alphavA, alphavB,
             idxrT, idxcT, avrT, avcT, hrowT, bevT, alphavT,
             normv, nsh, nst, nac,
             semA, semB, semHA, semHB, semT):
        c = lax.axis_index("c")
        s = lax.axis_index("s")
        wid = s * NC + c
        base = wid * EPW

        lane = jnp.arange(16, dtype=jnp.int32)
        e4 = lane >> 2
        hh = lane & 3
        masks = [e4 == j for j in range(4)]

        zero16 = jnp.zeros((16,), jnp.float32)

        def _z(i, _):
            normv[pl.ds(i * 16, 16)] = zero16
            return 0
        lax.fori_loop(0, NFLATP // 16, _z, 0)

        def fire(ci, idxr, idxc, bev, avr, avc, hrow, sem_av, sem_h, n):
            off = base + ci * CH
            pltpu.sync_copy(row_h.at[pl.ds(off, n)], idxr)
            pltpu.sync_copy(col_h.at[pl.ds(off, n)], idxc)
            pltpu.sync_copy(be_h.at[pl.ds(off * 4, n * 4)], bev)
            pltpu.async_copy(av16_h.at[idxr], avr, sem_av)
            pltpu.async_copy(av16_h.at[idxc], avc, sem_av)
            pltpu.async_copy(h_h.at[idxc], hrow, sem_h)

        def compute(idxr, avr, avc, bev, alphav, ngroups):
            def g_body(g, _):
                comb = jnp.zeros((16,), jnp.float32)
                for je in range(4):
                    e = g * 4 + je
                    vi = avr[e, :]
                    vj = avc[e, :]
                    s_e = _vgather(vi, hh) + _vgather(vj, hh + 4)
                    comb = jnp.where(masks[je], s_e, comb)
                raw = comb + bev[pl.ds(g * 16, 16)]
                raw = jnp.maximum(raw, raw * np.float32(0.2))
                al = jnp.exp(raw * BN_SCALE)
                alphav[pl.ds(g * 16, 16)] = al
                gi = g * 4 + e4
                rl = plsc.load_gather(idxr, [gi])
                nidx = (rl << 2) + hh
                for j in range(4):
                    plsc.addupdate_scatter(normv, [nidx], al, mask=masks[j])
                return 0
            lax.fori_loop(0, ngroups, g_body, 0)

        def drain(idxr, idxc, avr, avc, sem_av):
            pltpu.make_async_copy(av16_h.at[idxr], avr, sem_av).wait()
            pltpu.make_async_copy(av16_h.at[idxc], avc, sem_av).wait()

        def out(ci, idxc, hrow, alphav, sem_h, n):
            off = base + ci * CH
            pltpu.make_async_copy(h_h.at[idxc], hrow, sem_h).wait()
            pltpu.sync_copy(alphav, alpha_o.at[pl.ds(off * 4, n * 4)])
            pltpu.sync_copy(hrow, hcol_o.at[pl.ds(off, n)])

        bufA = (idxrA, idxcA, bevA, avrA, avcA, hrowA, semA, semHA)
        bufB = (idxrB, idxcB, bevB, avrB, avcB, hrowB, semB, semHB)
        fire(0, *bufA, CH)
        fire(1, *bufB, CH)

        def pair(j, _):
            for b, buf in ((0, bufA), (1, bufB)):
                idxr, idxc, bev, avr, avc, hrow, sem_av, sem_h = buf
                alphav = alphavA if b == 0 else alphavB
                ci = j * 2 + b
                drain(idxr, idxc, avr, avc, sem_av)
                compute(idxr, avr, avc, bev, alphav, CH // 4)
                out(ci, idxc, hrow, alphav, sem_h, CH)

                @pl.when(ci + 2 < NCHUNK)
                def _():
                    fire(ci + 2, idxr, idxc, bev, avr, avc, hrow, sem_av,
                         sem_h, CH)
            return 0
        lax.fori_loop(0, NPAIR, pair, 0)

        if TAIL:
            offT = base + NCHUNK * CH
            pltpu.sync_copy(row_h.at[pl.ds(offT, TAIL)], idxrT)
            pltpu.sync_copy(col_h.at[pl.ds(offT, TAIL)], idxcT)
            pltpu.sync_copy(be_h.at[pl.ds(offT * 4, TAIL * 4)], bevT)
            pltpu.async_copy(av16_h.at[idxrT], avrT, semT)
            pltpu.async_copy(av16_h.at[idxcT], avcT, semT)
            pltpu.async_copy(h_h.at[idxcT], hrowT, semT)
            pltpu.make_async_copy(av16_h.at[idxrT], avrT, semT).wait()
            pltpu.make_async_copy(av16_h.at[idxcT], avcT, semT).wait()
            pltpu.make_async_copy(h_h.at[idxcT], hrowT, semT).wait()
            compute(idxrT, avrT, avcT, bevT, alphavT, TAIL // 4)
            pltpu.sync_copy(alphavT, alpha_o.at[pl.ds(offT * 4, TAIL * 4)])
            pltpu.sync_copy(hrowT, hcol_o.at[pl.ds(offT, TAIL)])

        pltpu.sync_copy(normv, nsh.at[s])
        plsc.subcore_barrier()

        def _zn(i, _):
            nac[pl.ds(i * 16, 16)] = zero16
            return 0
        lax.fori_loop(0, RED // 16, _zn, 0)
        for t in range(NS):
            pltpu.sync_copy(nsh.at[t, pl.ds(s * RED, RED)], nst)

            def _acc(i, _):
                nac[pl.ds(i * 16, 16)] = (nac[pl.ds(i * 16, 16)]
                                          + nst[pl.ds(i * 16, 16)])
                return 0
            lax.fori_loop(0, RED // 16, _acc, 0)
        pltpu.sync_copy(nac, norm2_o.at[pl.ds(c * NFLATP + s * RED, RED)])

    kern = pl.kernel(
        body,
        out_type=[
            jax.ShapeDtypeStruct((E * 4,), jnp.float32),
            jax.ShapeDtypeStruct((E, HID), jnp.float32),
            jax.ShapeDtypeStruct((2 * NFLATP,), jnp.float32),
        ],
        mesh=_sc_mesh(),
        compiler_params=pltpu.CompilerParams(needs_layout_passes=False, use_tc_tiling_on_sc=False),
        scratch_types=[
            pltpu.VMEM((CH,), jnp.int32), pltpu.VMEM((CH,), jnp.int32),
            pltpu.VMEM((CH,), jnp.int32), pltpu.VMEM((CH,), jnp.int32),
            pltpu.VMEM((CH, 16), jnp.float32), pltpu.VMEM((CH, 16), jnp.float32),
            pltpu.VMEM((CH, 16), jnp.float32), pltpu.VMEM((CH, 16), jnp.float32),
            pltpu.VMEM((CH, HID), jnp.float32), pltpu.VMEM((CH, HID), jnp.float32),
            pltpu.VMEM((CH * 4,), jnp.float32), pltpu.VMEM((CH * 4,), jnp.float32),
            pltpu.VMEM((CH * 4,), jnp.float32), pltpu.VMEM((CH * 4,), jnp.float32),
            pltpu.VMEM((16,), jnp.int32), pltpu.VMEM((16,), jnp.int32),
            pltpu.VMEM((16, 16), jnp.float32), pltpu.VMEM((16, 16), jnp.float32),
            pltpu.VMEM((16, HID), jnp.float32), pltpu.VMEM((64,), jnp.float32),
            pltpu.VMEM((64,), jnp.float32),
            pltpu.VMEM((NFLATP,), jnp.float32),
            pltpu.VMEM_SHARED((NS, NFLATP), jnp.float32),
            pltpu.VMEM((NFLATP // NS,), jnp.float32),
            pltpu.VMEM((NFLATP // NS,), jnp.float32),
            pltpu.SemaphoreType.DMA, pltpu.SemaphoreType.DMA,
            pltpu.SemaphoreType.DMA, pltpu.SemaphoreType.DMA,
            pltpu.SemaphoreType.DMA,
        ],
        name="s1_alpha_norm",
    )
    return kern


def _build_s2(E, NFLATP):
    EPW = E // NW
    CH2 = 1000
    NCH = EPW // CH2
    STG = 4096

    def body(norm2_h, alpha_h, row_h, anorm_o, normv, nb, rowv, alpv, anv):
        c = lax.axis_index("c")
        s = lax.axis_index("s")
        wid = s * NC + c
        base = wid * EPW

        lane = jnp.arange(16, dtype=jnp.int32)
        e4 = lane >> 2
        hh = lane & 3

        pltpu.sync_copy(norm2_h.at[pl.ds(0, NFLATP)], normv)

        def stage(k, _):
            pltpu.sync_copy(norm2_h.at[pl.ds(NFLATP + k * STG, STG)], nb)

            def add(j, _):
                o = k * STG + j * 16
                normv[pl.ds(o, 16)] = (normv[pl.ds(o, 16)]
                                       + nb[pl.ds(j * 16, 16)])
                return 0
            lax.fori_loop(0, STG // 16, add, 0)
            return 0
        lax.fori_loop(0, NFLATP // STG, stage, 0)

        def chunk(ci, _):
            off = base + ci * CH2
            pltpu.sync_copy(row_h.at[pl.ds(off, CH2)], rowv)
            pltpu.sync_copy(alpha_h.at[pl.ds(off * 4, CH2 * 4)], alpv)

            def g_body(g, _):
                gi = g * 4 + e4
                rl = plsc.load_gather(rowv, [gi])
                nv = plsc.load_gather(normv, [(rl << 2) + hh])
                al = alpv[pl.ds(g * 16, 16)]
                anv[pl.ds(g * 16, 16)] = al / nv
                return 0
            lax.fori_loop(0, CH2 // 4, g_body, 0)
            pltpu.sync_copy(anv, anorm_o.at[pl.ds(off * 4, CH2 * 4)])
            return 0
        lax.fori_loop(0, NCH, chunk, 0)

    return pl.kernel(
        body,
        out_type=jax.ShapeDtypeStruct((E * 4,), jnp.float32),
        mesh=_sc_mesh(),
        compiler_params=pltpu.CompilerParams(needs_layout_passes=False, use_tc_tiling_on_sc=False),
        scratch_types=[
            pltpu.VMEM((NFLATP,), jnp.float32),
            pltpu.VMEM((STG,), jnp.float32),
            pltpu.VMEM((CH2,), jnp.int32),
            pltpu.VMEM((CH2 * 4,), jnp.float32),
            pltpu.VMEM((CH2 * 4,), jnp.float32),
        ],
        name="s2_anorm",
    )


def _build_s3(E, NP):
    EPW = E // NW
    NCHUNK = EPW // CH
    NPAIR = NCHUNK // 2
    TAIL = EPW - NCHUNK * CH
    ROWS_PER_TILE = NP // NS

    def body(sp_h, row_h, acc2_o,
             idx0, idx1, rows0, rows1, idxT, rowsT, zb,
             acc_sh, sem0, sem1, semT):
        c = lax.axis_index("c")
        s = lax.axis_index("s")
        wid = s * NC + c
        base = wid * EPW

        zero16 = jnp.zeros((16,), jnp.float32)

        def _z(i, _):
            r = i >> 2
            q = i & 3
            zb[r, pl.ds(q * 16, 16)] = zero16
            return 0
        lax.fori_loop(0, CH * HID // 16, _z, 0)
        for k in range(ROWS_PER_TILE // CH):
            pltpu.sync_copy(zb,
                            acc_sh.at[pl.ds(s * ROWS_PER_TILE + k * CH, CH)])
        plsc.subcore_barrier()

        def fire(ci, idx, rows, sem):
            off = base + ci * CH
            pltpu.sync_copy(row_h.at[pl.ds(off, CH)], idx)
            pltpu.async_copy(sp_h.at[pl.ds(off, CH)], rows, sem)

        fire(0, idx0, rows0, sem0)
        fire(1, idx1, rows1, sem1)

        def pair(j, _):
            for b, (idx, rows, sem) in ((0, (idx0, rows0, sem0)),
                                        (1, (idx1, rows1, sem1))):
                ci = j * 2 + b
                pltpu.make_async_copy(sp_h.at[pl.ds(base, CH)], rows,
                                      sem).wait()
                pltpu.sync_copy(rows, acc_sh.at[idx], add=True)

                @pl.when(ci + 2 < NCHUNK)
                def _():
                    fire(ci + 2, idx, rows, sem)
            return 0
        lax.fori_loop(0, NPAIR, pair, 0)

        if TAIL:
            offT = base + NCHUNK * CH
            pltpu.sync_copy(row_h.at[pl.ds(offT, TAIL)], idxT)
            pltpu.async_copy(sp_h.at[pl.ds(offT, TAIL)], rowsT, semT).wait()
            pltpu.sync_copy(rowsT, acc_sh.at[idxT], add=True)

        plsc.subcore_barrier()
        pltpu.sync_copy(
            acc_sh.at[pl.ds(s * ROWS_PER_TILE, ROWS_PER_TILE)],
            acc2_o.at[pl.ds(c * NP + s * ROWS_PER_TILE, ROWS_PER_TILE)])

    return pl.kernel(
        body,
        out_type=jax.ShapeDtypeStruct((2 * NP, HID), jnp.float32),
        mesh=_sc_mesh(),
        compiler_params=pltpu.CompilerParams(needs_layout_passes=False, use_tc_tiling_on_sc=False),
        scratch_types=[
            pltpu.VMEM((CH,), jnp.int32), pltpu.VMEM((CH,), jnp.int32),
            pltpu.VMEM((CH, HID), jnp.float32),
            pltpu.VMEM((CH, HID), jnp.float32),
            pltpu.VMEM((16,), jnp.int32), pltpu.VMEM((16, HID), jnp.float32),
            pltpu.VMEM((CH, HID), jnp.float32),
            pltpu.VMEM_SHARED((NP, HID), jnp.float32),
            pltpu.SemaphoreType.DMA, pltpu.SemaphoreType.DMA,
            pltpu.SemaphoreType.DMA,
        ],
        name="s3_scatter",
    )


def kernel(x, edge_index, edge_attr, batch_idx, global_features, node_W,
           node_b, edge_W, edge_b, conv_W, conv_att, conv_bias, ga_W0, ga_b0,
           ga_W1, ga_b1, ga_W2, ga_b2, out_W0, out_b0, out_W1, out_b1):
    N, DF = x.shape
    E = edge_index.shape[1]
    G, GD = global_features.shape
    BN = 1024
    NP = ((N + BN - 1) // BN) * BN
    NFLATP = NP * HEADS
    BE = 2560
    QB = BE // 4
    Q = E // 4

    f32 = jnp.float32
    row = edge_index[0]
    col = edge_index[1]
    x_p = jnp.pad(x, ((0, NP - N), (0, 0)))
    bidx3 = jnp.pad(batch_idx, (0, NP - N)).reshape(NP // BN, 1, BN)

    Wt = conv_W[:, :HID, :]
    Wb = conv_W[:, HID:, :]
    att = conv_att[:, 0]
    att_i = att[..., :HID]
    att_j = att[..., HID:]
    Wt4 = Wt.reshape(LAYERS, HID, HEADS, HID)
    Wb4 = Wb.reshape(LAYERS, HID, HEADS, HID)
    Ai = jnp.einsum('lkhd,lhd->lkh', Wt4, att_i)
    Aj = jnp.einsum('lkhd,lhd->lkh', Wt4, att_j)
    Bij = jnp.einsum('lkhd,lhd->lkh', Wb4, att_i + att_j)
    AiAj = jnp.concatenate([Ai, Aj], axis=2)
    BijAll = jnp.transpose(Bij, (1, 0, 2)).reshape(HID, LAYERS * HEADS)

    p0 = pl.pallas_call(
        _p0_body,
        grid=(E // BE,),
        in_specs=[
            pl.BlockSpec((BE, 16), lambda i: (i, 0)),
            _full((16, HID)), _full((1, HID)), _full((HID, LAYERS * HEADS)),
        ],
        out_specs=[pl.BlockSpec((BE, HID), lambda i: (i, 0))]
        + [pl.BlockSpec((BE, HEADS), lambda i: (i, 0))] * LAYERS,
        out_shape=[jax.ShapeDtypeStruct((E, HID), f32)]
        + [jax.ShapeDtypeStruct((E, HEADS), f32)] * LAYERS,
    )
    def _eperm(a):
        return a.reshape(E // BE, BE // 4, 4, *a.shape[1:]).swapaxes(1, 2)\
            .reshape(a.shape)

    rowp = _eperm(row)
    colp = _eperm(col)
    ea, *be_l = p0(_eperm(edge_attr), edge_W, edge_b.reshape(1, HID), BijAll)
    be_flat = [b.reshape(E * 4) for b in be_l]

    h = pl.pallas_call(
        _p1_body,
        grid=(NP // BN,),
        in_specs=[pl.BlockSpec((BN, DF), lambda i: (i, 0)),
                  _full((DF, HID)), _full((1, HID))],
        out_specs=pl.BlockSpec((BN, HID), lambda i: (i, 0)),
        out_shape=jax.ShapeDtypeStruct((NP, HID), f32),
    )(x_p, node_W, node_b.reshape(1, HID))

    s1 = _build_s1(E, NP, NFLATP)
    s2 = _build_s2(E, NFLATP)
    s3 = _build_s3(E, NP)

    t1a = pl.pallas_call(
        _t1a_body,
        grid=(NP // BN,),
        in_specs=[pl.BlockSpec((BN, HID), lambda i: (i, 0)), _full((HID, 8))],
        out_specs=pl.BlockSpec((BN, 16), lambda i: (i, 0)),
        out_shape=jax.ShapeDtypeStruct((NP, 16), f32),
    )
    t1 = pl.pallas_call(
        _t1_body,
        grid=(NP // BN,),
        in_specs=[pl.BlockSpec((BN, HID), lambda i: (i, 0)),
                  pl.BlockSpec((2, BN, HID), lambda i: (0, i, 0)),
                  _full((1, HID)), _full((HID, 8))],
        out_specs=[pl.BlockSpec((BN, HID), lambda i: (i, 0)),
                   pl.BlockSpec((BN, 16), lambda i: (i, 0))],
        out_shape=[jax.ShapeDtypeStruct((NP, HID), f32),
                   jax.ShapeDtypeStruct((NP, 16), f32)],
    )
    t2 = pl.pallas_call(
        _t2_body,
        grid=(E // BE,),
        in_specs=[pl.BlockSpec((BE, HID), lambda i: (i, 0)),
                  pl.BlockSpec((BE, HID), lambda i: (i, 0)),
                  pl.BlockSpec((BE, HEADS), lambda i: (i, 0)),
                  _full((2 * HID, HEADS * HID))],
        out_specs=pl.BlockSpec((HEADS, QB, HID), lambda i: (0, i, 0)),
        out_shape=jax.ShapeDtypeStruct((HEADS, Q, HID), f32),
    )

    acc2 = None
    for l in range(LAYERS):
        if l == 0:
            av16 = t1a(h, AiAj[0])
        else:
            h, av16 = t1(h, acc2.reshape(2, NP, HID),
                         conv_bias[l - 1].reshape(1, HID), AiAj[l])
        alpha, hcol, norm2 = s1(av16, h, rowp, colp, be_flat[l])
        anorm = s2(norm2, alpha, rowp)
        S = t2(hcol, ea, anorm.reshape(E, HEADS), conv_W[l])
        acc2 = s3(S.reshape(E, HID), row)

    tail1 = pl.pallas_call(
        _make_tail1_body(N),
        grid=(NP // BN,),
        in_specs=[pl.BlockSpec((BN, HID), lambda i: (i, 0)),
                  pl.BlockSpec((2, BN, HID), lambda i: (0, i, 0)),
                  _full((1, HID)),
                  pl.BlockSpec((1, 1, BN), lambda i: (i, 0, 0)),
                  _full((G, GD)), _full((HID, HID)), _full((GD, HID)),
                  _full((1, HID)), _full((HID, HID)), _full((1, HID)),
                  _full((HID, 1)), _full((1, 1))],
        out_specs=[pl.BlockSpec((BN, HID), lambda i: (i, 0)),
                   pl.BlockSpec((BN, 1), lambda i: (i, 0)),
                   pl.BlockSpec((G, 1), lambda i: (0, 0))],
        out_shape=[jax.ShapeDtypeStruct((NP, HID), f32),
                   jax.ShapeDtypeStruct((NP, 1), f32),
                   jax.ShapeDtypeStruct((G, 1), f32)],
    )
    h5, attn, normg = tail1(
        h, acc2.reshape(2, NP, HID), conv_bias[LAYERS - 1].reshape(1, HID),
        bidx3, global_features, ga_W0[:HID], ga_W0[HID:],
        ga_b0.reshape(1, HID), ga_W1, ga_b1.reshape(1, HID), ga_W2,
        ga_b2.reshape(1, 1))

    tail2 = pl.pallas_call(
        _tail2_body,
        grid=(NP // BN,),
        in_specs=[pl.BlockSpec((BN, HID), lambda i: (i, 0)),
                  pl.BlockSpec((BN, 1), lambda i: (i, 0)),
                  _full((G, 1)),
                  pl.BlockSpec((1, 1, BN), lambda i: (i, 0, 0)),
                  _full((HID, HID)), _full((1, HID)),
                  _full((HID, 1)), _full((1, 1))],
        out_specs=[pl.BlockSpec((G, HID), lambda i: (0, 0)),
                   pl.BlockSpec((G, 1), lambda i: (0, 0))],
        out_shape=[jax.ShapeDtypeStruct((G, HID), f32),
                   jax.ShapeDtypeStruct((G, 1), f32)],
    )
    _, o = tail2(h5, attn, normg, bidx3, out_W0, out_b0.reshape(1, HID),
                 out_W1, out_b1.reshape(1, 1))
    return o.reshape(G)


# speedup vs baseline: 4.4510x; 1.0867x over previous
import functools

import jax
import jax.numpy as jnp
import numpy as np
from jax import lax
from jax.experimental import pallas as pl
from jax.experimental.pallas import tpu as pltpu
from jax.experimental.pallas import tpu_sc as plsc

HID = 64
HEADS = 4
LAYERS = 5
BN_SCALE = np.float32(1.0 / np.sqrt(1.0 + 1e-3))

NC = 2
NS = 16
NW = NC * NS

CH = 128


def _lrelu(t):
    return jnp.maximum(t, t * np.float32(0.2))


def _p0_body(eattr, ew, eb, bij, ea_o, *be_o):
    t = jnp.dot(eattr[...], ew[...], preferred_element_type=jnp.float32)
    t = _lrelu(t + eb[...])
    ea_o[...] = t
    ball = jnp.dot(t, bij[...], preferred_element_type=jnp.float32)
    for l in range(LAYERS):
        be_o[l][...] = ball[:, 4 * l:4 * l + 4]


def _p1_body(x, nw, nb, h_o):
    t = jnp.dot(x[...], nw[...], preferred_element_type=jnp.float32)
    h_o[...] = _lrelu(t + nb[...])


def _t1a_body(h, aiaj, av_o):
    av = jnp.dot(h[...], aiaj[...], preferred_element_type=jnp.float32)
    av_o[...] = jnp.concatenate([av, jnp.zeros_like(av)], axis=1)


def _t1_body(h, acc2, bias, aiaj, h_o, av_o):
    hn = h[...] + (acc2[0] + acc2[1]) * np.float32(0.25) + bias[...]
    h_o[...] = hn
    av = jnp.dot(hn, aiaj[...], preferred_element_type=jnp.float32)
    av_o[...] = jnp.concatenate([av, jnp.zeros_like(av)], axis=1)


def _t2_body(hcol, ea, anorm, w, s_o):
    an = anorm[...]
    xj = jnp.dot(jnp.concatenate([hcol[...], ea[...]], axis=1), w[...],
                 preferred_element_type=jnp.float32)
    bq = s_o.shape[1]
    for h in range(HEADS):
        m = xj[:, HID * h:HID * h + HID] * an[:, h:h + 1]
        s_o[h, :, :] = (m[0:bq] + m[bq:2 * bq] + m[2 * bq:3 * bq]
                        + m[3 * bq:4 * bq])


def _make_tail1_body(n_real):
    def body(h, acc2, bias, bidx, gf, w0h, w0g, b0, w1, b1, w2, b2,
             h5_o, attn_o, normg_o):
        i = pl.program_id(0)
        bn = h.shape[0]
        h5 = h[...] + (acc2[0] + acc2[1]) * np.float32(0.25) + bias[...]
        h5_o[...] = h5
        bi = bidx[0, 0, :]
        oh = (bi[:, None] == lax.broadcasted_iota(
            jnp.int32, (bn, gf.shape[0]), 1)).astype(jnp.float32)
        gs = jnp.dot(gf[...], w0g[...], preferred_element_type=jnp.float32)
        a0 = jnp.dot(h5, w0h[...], preferred_element_type=jnp.float32) \
            + jnp.dot(oh, gs, preferred_element_type=jnp.float32) + b0[...]
        a0 = jnp.maximum(a0 * BN_SCALE, 0.0)
        a1 = jnp.dot(a0, w1[...], preferred_element_type=jnp.float32) + b1[...]
        a1 = jnp.maximum(a1 * BN_SCALE, 0.0)
        lg = jnp.dot(a1, w2[...], preferred_element_type=jnp.float32) + b2[...]
        at = jnp.exp(lg)
        grow = i * bn + lax.broadcasted_iota(jnp.int32, (bn, 1), 0)
        at = jnp.where(grow < n_real, at, 0.0)
        attn_o[...] = at

        @pl.when(i == 0)
        def _():
            normg_o[...] = jnp.zeros_like(normg_o)
        normg_o[...] += lax.dot_general(oh, at, (((0,), (0,)), ((), ())),
                                        preferred_element_type=jnp.float32)
    return body


def _tail2_body(h5, attn, normg, bidx, ow0, ob0, ow1, ob1, pooled_o, o_o):
    i = pl.program_id(0)
    bn = h5.shape[0]
    bi = bidx[0, 0, :]
    oh = (bi[:, None] == lax.broadcasted_iota(
        jnp.int32, (bn, normg.shape[0]), 1)).astype(jnp.float32)
    nrm = jnp.dot(oh, normg[...], preferred_element_type=jnp.float32)
    atn = attn[...] / nrm
    hw = h5[...] * atn

    @pl.when(i == 0)
    def _():
        pooled_o[...] = jnp.zeros_like(pooled_o)
    pooled_o[...] += lax.dot_general(oh, hw, (((0,), (0,)), ((), ())),
                                     preferred_element_type=jnp.float32)

    @pl.when(i == pl.num_programs(0) - 1)
    def _():
        p = pooled_o[...]
        t0 = jnp.maximum(
            jnp.dot(p, ow0[...], preferred_element_type=jnp.float32) + ob0[...],
            0.0)
        o_o[...] = jnp.dot(t0, ow1[...],
                           preferred_element_type=jnp.float32) + ob1[...]


def _full(shape):
    return pl.BlockSpec(shape, lambda i: (0,) * len(shape))


_GDN = lax.GatherDimensionNumbers(offset_dims=(), collapsed_slice_dims=(0,),
                                  start_index_map=(0,))


def _vgather(v, idx):
    return lax.gather(v, idx[:, None], _GDN, (1,),
                      mode=lax.GatherScatterMode.PROMISE_IN_BOUNDS)


def _sc_mesh():
    return plsc.VectorSubcoreMesh(core_axis_name="c", subcore_axis_name="s")


def _build_s1(E, NP, NFLATP):
    EPW = E // NW
    SUPER = 2000
    NSUPER = EPW // SUPER
    SUB = 80
    NSUB = SUPER // SUB
    NPAIR = NSUB // 2
    RED = NFLATP // NS

    def body(av16_h, h_h, row_h, col_h, be_h,
             alpha_o, hcol_o, norm2_o,
             idxrS, idxcS, bevS, alphavS,
             avrA, avcA, avrB, avcB, hrowA, hrowB,
             normv, nsh, nst, nac,
             semAVA, semAVB, semHA, semHB, semOA, semOB):
        c = lax.axis_index("c")
        s = lax.axis_index("s")
        wid = s * NC + c
        base = wid * EPW

        lane = jnp.arange(16, dtype=jnp.int32)
        e4 = lane >> 2
        hh = lane & 3
        masks = [e4 == j for j in range(4)]

        zero16 = jnp.zeros((16,), jnp.float32)

        def _z(i, _):
            normv[pl.ds(i * 16, 16)] = zero16
            return 0
        lax.fori_loop(0, NFLATP // 16, _z, 0)

        bufA = (avrA, avcA, hrowA, semAVA, semHA, semOA)
        bufB = (avrB, avcB, hrowB, semAVB, semHB, semOB)

        def super_body(sp, _):
            soff = base + sp * SUPER
            pltpu.sync_copy(row_h.at[pl.ds(soff, SUPER)], idxrS)
            pltpu.sync_copy(col_h.at[pl.ds(soff, SUPER)], idxcS)
            pltpu.sync_copy(be_h.at[pl.ds(soff * 4, SUPER * 4)], bevS)

            def fire(sub, avr, avc, hrow, sem_av, sem_h):
                ir = idxrS.at[pl.ds(sub * SUB, SUB)]
                ic = idxcS.at[pl.ds(sub * SUB, SUB)]
                pltpu.async_copy(av16_h.at[ir], avr, sem_av)
                pltpu.async_copy(av16_h.at[ic], avc, sem_av)
                pltpu.async_copy(h_h.at[ic], hrow, sem_h)

            def waitg(sub, avr, avc, hrow, sem_av, sem_h):
                ir = idxrS.at[pl.ds(sub * SUB, SUB)]
                ic = idxcS.at[pl.ds(sub * SUB, SUB)]
                pltpu.make_async_copy(av16_h.at[ir], avr, sem_av).wait()
                pltpu.make_async_copy(av16_h.at[ic], avc, sem_av).wait()
                pltpu.make_async_copy(h_h.at[ic], hrow, sem_h).wait()

            def compute(sub, avr, avc):
                sb = sub * SUB

                def g_body(g, _):
                    comb = jnp.zeros((16,), jnp.float32)
                    for je in range(4):
                        e = g * 4 + je
                        vi = avr[e, :]
                        vj = avc[e, :]
                        s_e = _vgather(vi, hh) + _vgather(vj, hh + 4)
                        comb = jnp.where(masks[je], s_e, comb)
                    raw = comb + bevS[pl.ds((sb + g * 4) * 4, 16)]
                    raw = jnp.maximum(raw, raw * np.float32(0.2))
                    al = jnp.exp(raw * BN_SCALE)
                    alphavS[pl.ds((sb + g * 4) * 4, 16)] = al
                    gi = sb + g * 4 + e4
                    rl = plsc.load_gather(idxrS, [gi])
                    nidx = (rl << 2) + hh
                    for j in range(4):
                        plsc.addupdate_scatter(normv, [nidx], al,
                                               mask=masks[j])
                    return 0
                lax.fori_loop(0, SUB // 4, g_body, 0)

            def out_hcol(sub, hrow, sem_o):
                dst = hcol_o.at[pl.ds(soff + sub * SUB, SUB)]
                pltpu.async_copy(hrow, dst, sem_o)

            def wait_out(sub, hrow, sem_o):
                dst = hcol_o.at[pl.ds(soff + sub * SUB, SUB)]
                pltpu.make_async_copy(hrow, dst, sem_o).wait()

            fire(0, *bufA[:3], bufA[3], bufA[4])
            fire(1, *bufB[:3], bufB[3], bufB[4])

            def pair(j, _):
                for b, (avr, avc, hrow, sem_av, sem_h, sem_o) in (
                        (0, bufA), (1, bufB)):
                    sub = j * 2 + b
                    waitg(sub, avr, avc, hrow, sem_av, sem_h)
                    out_hcol(sub, hrow, sem_o)
                    compute(sub, avr, avc)
                    wait_out(sub, hrow, sem_o)

                    @pl.when(sub + 2 < NSUB)
                    def _():
                        fire(sub + 2, avr, avc, hrow, sem_av, sem_h)
                return 0
            lax.fori_loop(0, NPAIR, pair, 0)

            if NSUB % 2:
                sub = NSUB - 1
                avr, avc, hrow, sem_av, sem_h, sem_o = bufA
                waitg(sub, avr, avc, hrow, sem_av, sem_h)
                out_hcol(sub, hrow, sem_o)
                compute(sub, avr, avc)
                wait_out(sub, hrow, sem_o)

            pltpu.sync_copy(alphavS, alpha_o.at[pl.ds(soff * 4, SUPER * 4)])
            return 0
        lax.fori_loop(0, NSUPER, super_body, 0)

        pltpu.sync_copy(normv, nsh.at[s])
        plsc.subcore_barrier()

        def _zn(i, _):
            nac[pl.ds(i * 16, 16)] = zero16
            return 0
        lax.fori_loop(0, RED // 16, _zn, 0)
        for t in range(NS):
            pltpu.sync_copy(nsh.at[t, pl.ds(s * RED, RED)], nst)

            def _acc(i, _):
                nac[pl.ds(i * 16, 16)] = (nac[pl.ds(i * 16, 16)]
                                          + nst[pl.ds(i * 16, 16)])
                return 0
            lax.fori_loop(0, RED // 16, _acc, 0)
        pltpu.sync_copy(nac, norm2_o.at[pl.ds(c * NFLATP + s * RED, RED)])

    kern = pl.kernel(
        body,
        out_type=[
            jax.ShapeDtypeStruct((E * 4,), jnp.float32),
            jax.ShapeDtypeStruct((E, HID), jnp.float32),
            jax.ShapeDtypeStruct((2 * NFLATP,), jnp.float32),
        ],
        mesh=_sc_mesh(),
        compiler_params=pltpu.CompilerParams(needs_layout_passes=False, use_tc_tiling_on_sc=False),
        scratch_types=[
            pltpu.VMEM((SUPER,), jnp.int32), pltpu.VMEM((SUPER,), jnp.int32),
            pltpu.VMEM((SUPER * 4,), jnp.float32),
            pltpu.VMEM((SUPER * 4,), jnp.float32),
            pltpu.VMEM((SUB, 16), jnp.float32), pltpu.VMEM((SUB, 16), jnp.float32),
            pltpu.VMEM((SUB, 16), jnp.float32), pltpu.VMEM((SUB, 16), jnp.float32),
            pltpu.VMEM((SUB, HID), jnp.float32), pltpu.VMEM((SUB, HID), jnp.float32),
            pltpu.VMEM((NFLATP,), jnp.float32),
            pltpu.VMEM_SHARED((NS, NFLATP), jnp.float32),
            pltpu.VMEM((NFLATP // NS,), jnp.float32),
            pltpu.VMEM((NFLATP // NS,), jnp.float32),
            pltpu.SemaphoreType.DMA, pltpu.SemaphoreType.DMA,
            pltpu.SemaphoreType.DMA, pltpu.SemaphoreType.DMA,
            pltpu.SemaphoreType.DMA, pltpu.SemaphoreType.DMA,
        ],
        name="s1_alpha_norm",
    )
    return kern


def _build_s2(E, NFLATP):
    EPW = E // NW
    CH2 = 2000
    NCH = EPW // CH2
    STG = 4096

    def body(norm2_h, alpha_h, row_h, anorm_o, normv, nb, rowv, alpv, anv):
        c = lax.axis_index("c")
        s = lax.axis_index("s")
        wid = s * NC + c
        base = wid * EPW

        lane = jnp.arange(16, dtype=jnp.int32)
        e4 = lane >> 2
        hh = lane & 3

        pltpu.sync_copy(norm2_h.at[pl.ds(0, NFLATP)], normv)

        def stage(k, _):
            pltpu.sync_copy(norm2_h.at[pl.ds(NFLATP + k * STG, STG)], nb)

            def add(j, _):
                o = k * STG + j * 16
                normv[pl.ds(o, 16)] = (normv[pl.ds(o, 16)]
                                       + nb[pl.ds(j * 16, 16)])
                return 0
            lax.fori_loop(0, STG // 16, add, 0)
            return 0
        lax.fori_loop(0, NFLATP // STG, stage, 0)

        def chunk(ci, _):
            off = base + ci * CH2
            pltpu.sync_copy(row_h.at[pl.ds(off, CH2)], rowv)
            pltpu.sync_copy(alpha_h.at[pl.ds(off * 4, CH2 * 4)], alpv)

            def g_body(g, _):
                gi = g * 4 + e4
                rl = plsc.load_gather(rowv, [gi])
                nv = plsc.load_gather(normv, [(rl << 2) + hh])
                al = alpv[pl.ds(g * 16, 16)]
                anv[pl.ds(g * 16, 16)] = al / nv
                return 0
            lax.fori_loop(0, CH2 // 4, g_body, 0)
            pltpu.sync_copy(anv, anorm_o.at[pl.ds(off * 4, CH2 * 4)])
            return 0
        lax.fori_loop(0, NCH, chunk, 0)

    return pl.kernel(
        body,
        out_type=jax.ShapeDtypeStruct((E * 4,), jnp.float32),
        mesh=_sc_mesh(),
        compiler_params=pltpu.CompilerParams(needs_layout_passes=False, use_tc_tiling_on_sc=False),
        scratch_types=[
            pltpu.VMEM((NFLATP,), jnp.float32),
            pltpu.VMEM((STG,), jnp.float32),
            pltpu.VMEM((CH2,), jnp.int32),
            pltpu.VMEM((CH2 * 4,), jnp.float32),
            pltpu.VMEM((CH2 * 4,), jnp.float32),
        ],
        name="s2_anorm",
    )


def _build_s3(E, NP):
    EPW = E // NW
    NCHUNK = EPW // CH
    NPAIR = NCHUNK // 2
    TAIL = EPW - NCHUNK * CH
    ROWS_PER_TILE = NP // NS

    def body(sp_h, row_h, acc2_o,
             idx0, idx1, rows0, rows1, idxT, rowsT, zb,
             acc_sh, sem0, sem1, semT):
        c = lax.axis_index("c")
        s = lax.axis_index("s")
        wid = s * NC + c
        base = wid * EPW

        zero16 = jnp.zeros((16,), jnp.float32)

        def _z(i, _):
            r = i >> 2
            q = i & 3
            zb[r, pl.ds(q * 16, 16)] = zero16
            return 0
        lax.fori_loop(0, CH * HID // 16, _z, 0)
        for k in range(ROWS_PER_TILE // CH):
            pltpu.sync_copy(zb,
                            acc_sh.at[pl.ds(s * ROWS_PER_TILE + k * CH, CH)])
        plsc.subcore_barrier()

        def fire(ci, idx, rows, sem):
            off = base + ci * CH
            pltpu.async_copy(row_h.at[pl.ds(off, CH)], idx, sem)
            pltpu.async_copy(sp_h.at[pl.ds(off, CH)], rows, sem)

        fire(0, idx0, rows0, sem0)
        fire(1, idx1, rows1, sem1)

        def pair(j, _):
            for b, (idx, rows, sem) in ((0, (idx0, rows0, sem0)),
                                        (1, (idx1, rows1, sem1))):
                ci = j * 2 + b
                pltpu.make_async_copy(row_h.at[pl.ds(base, CH)], idx,
                                      sem).wait()
                pltpu.make_async_copy(sp_h.at[pl.ds(base, CH)], rows,
                                      sem).wait()
                pltpu.sync_copy(rows, acc_sh.at[idx], add=True)

                @pl.when(ci + 2 < NCHUNK)
                def _():
                    fire(ci + 2, idx, rows, sem)
            return 0
        lax.fori_loop(0, NPAIR, pair, 0)

        if TAIL:
            offT = base + NCHUNK * CH
            pltpu.sync_copy(row_h.at[pl.ds(offT, TAIL)], idxT)
            pltpu.async_copy(sp_h.at[pl.ds(offT, TAIL)], rowsT, semT).wait()
            pltpu.sync_copy(rowsT, acc_sh.at[idxT], add=True)

        plsc.subcore_barrier()
        pltpu.sync_copy(
            acc_sh.at[pl.ds(s * ROWS_PER_TILE, ROWS_PER_TILE)],
            acc2_o.at[pl.ds(c * NP + s * ROWS_PER_TILE, ROWS_PER_TILE)])

    return pl.kernel(
        body,
        out_type=jax.ShapeDtypeStruct((2 * NP, HID), jnp.float32),
        mesh=_sc_mesh(),
        compiler_params=pltpu.CompilerParams(needs_layout_passes=False, use_tc_tiling_on_sc=False),
        scratch_types=[
            pltpu.VMEM((CH,), jnp.int32), pltpu.VMEM((CH,), jnp.int32),
            pltpu.VMEM((CH, HID), jnp.float32),
            pltpu.VMEM((CH, HID), jnp.float32),
            pltpu.VMEM((16,), jnp.int32), pltpu.VMEM((16, HID), jnp.float32),
            pltpu.VMEM((CH, HID), jnp.float32),
            pltpu.VMEM_SHARED((NP, HID), jnp.float32),
            pltpu.SemaphoreType.DMA, pltpu.SemaphoreType.DMA,
            pltpu.SemaphoreType.DMA,
        ],
        name="s3_scatter",
    )


def kernel(x, edge_index, edge_attr, batch_idx, global_features, node_W,
           node_b, edge_W, edge_b, conv_W, conv_att, conv_bias, ga_W0, ga_b0,
           ga_W1, ga_b1, ga_W2, ga_b2, out_W0, out_b0, out_W1, out_b1):
    N, DF = x.shape
    E = edge_index.shape[1]
    G, GD = global_features.shape
    BN = 1024
    NP = ((N + BN - 1) // BN) * BN
    NFLATP = NP * HEADS
    BE = 2560
    QB = BE // 4
    Q = E // 4

    f32 = jnp.float32
    row = edge_index[0]
    col = edge_index[1]
    x_p = jnp.pad(x, ((0, NP - N), (0, 0)))
    bidx3 = jnp.pad(batch_idx, (0, NP - N)).reshape(NP // BN, 1, BN)

    Wt = conv_W[:, :HID, :]
    Wb = conv_W[:, HID:, :]
    att = conv_att[:, 0]
    att_i = att[..., :HID]
    att_j = att[..., HID:]
    Wt4 = Wt.reshape(LAYERS, HID, HEADS, HID)
    Wb4 = Wb.reshape(LAYERS, HID, HEADS, HID)
    Ai = jnp.einsum('lkhd,lhd->lkh', Wt4, att_i)
    Aj = jnp.einsum('lkhd,lhd->lkh', Wt4, att_j)
    Bij = jnp.einsum('lkhd,lhd->lkh', Wb4, att_i + att_j)
    AiAj = jnp.concatenate([Ai, Aj], axis=2)
    BijAll = jnp.transpose(Bij, (1, 0, 2)).reshape(HID, LAYERS * HEADS)

    p0 = pl.pallas_call(
        _p0_body,
        grid=(E // BE,),
        in_specs=[
            pl.BlockSpec((BE, 16), lambda i: (i, 0)),
            _full((16, HID)), _full((1, HID)), _full((HID, LAYERS * HEADS)),
        ],
        out_specs=[pl.BlockSpec((BE, HID), lambda i: (i, 0))]
        + [pl.BlockSpec((BE, HEADS), lambda i: (i, 0))] * LAYERS,
        out_shape=[jax.ShapeDtypeStruct((E, HID), f32)]
        + [jax.ShapeDtypeStruct((E, HEADS), f32)] * LAYERS,
    )
    def _eperm(a):
        return a.reshape(E // BE, BE // 4, 4, *a.shape[1:]).swapaxes(1, 2)\
            .reshape(a.shape)

    rowp = _eperm(row)
    colp = _eperm(col)
    ea, *be_l = p0(_eperm(edge_attr), edge_W, edge_b.reshape(1, HID), BijAll)
    be_flat = [b.reshape(E * 4) for b in be_l]

    h = pl.pallas_call(
        _p1_body,
        grid=(NP // BN,),
        in_specs=[pl.BlockSpec((BN, DF), lambda i: (i, 0)),
                  _full((DF, HID)), _full((1, HID))],
        out_specs=pl.BlockSpec((BN, HID), lambda i: (i, 0)),
        out_shape=jax.ShapeDtypeStruct((NP, HID), f32),
    )(x_p, node_W, node_b.reshape(1, HID))

    s1 = _build_s1(E, NP, NFLATP)
    s2 = _build_s2(E, NFLATP)
    s3 = _build_s3(E, NP)

    t1a = pl.pallas_call(
        _t1a_body,
        grid=(NP // BN,),
        in_specs=[pl.BlockSpec((BN, HID), lambda i: (i, 0)), _full((HID, 8))],
        out_specs=pl.BlockSpec((BN, 16), lambda i: (i, 0)),
        out_shape=jax.ShapeDtypeStruct((NP, 16), f32),
    )
    t1 = pl.pallas_call(
        _t1_body,
        grid=(NP // BN,),
        in_specs=[pl.BlockSpec((BN, HID), lambda i: (i, 0)),
                  pl.BlockSpec((2, BN, HID), lambda i: (0, i, 0)),
                  _full((1, HID)), _full((HID, 8))],
        out_specs=[pl.BlockSpec((BN, HID), lambda i: (i, 0)),
                   pl.BlockSpec((BN, 16), lambda i: (i, 0))],
        out_shape=[jax.ShapeDtypeStruct((NP, HID), f32),
                   jax.ShapeDtypeStruct((NP, 16), f32)],
    )
    t2 = pl.pallas_call(
        _t2_body,
        grid=(E // BE,),
        in_specs=[pl.BlockSpec((BE, HID), lambda i: (i, 0)),
                  pl.BlockSpec((BE, HID), lambda i: (i, 0)),
                  pl.BlockSpec((BE, HEADS), lambda i: (i, 0)),
                  _full((2 * HID, HEADS * HID))],
        out_specs=pl.BlockSpec((HEADS, QB, HID), lambda i: (0, i, 0)),
        out_shape=jax.ShapeDtypeStruct((HEADS, Q, HID), f32),
    )

    acc2 = None
    for l in range(LAYERS):
        if l == 0:
            av16 = t1a(h, AiAj[0])
        else:
            h, av16 = t1(h, acc2.reshape(2, NP, HID),
                         conv_bias[l - 1].reshape(1, HID), AiAj[l])
        alpha, hcol, norm2 = s1(av16, h, rowp, colp, be_flat[l])
        anorm = s2(norm2, alpha, rowp)
        S = t2(hcol, ea, anorm.reshape(E, HEADS), conv_W[l])
        acc2 = s3(S.reshape(E, HID), row)

    tail1 = pl.pallas_call(
        _make_tail1_body(N),
        grid=(NP // BN,),
        in_specs=[pl.BlockSpec((BN, HID), lambda i: (i, 0)),
                  pl.BlockSpec((2, BN, HID), lambda i: (0, i, 0)),
                  _full((1, HID)),
                  pl.BlockSpec((1, 1, BN), lambda i: (i, 0, 0)),
                  _full((G, GD)), _full((HID, HID)), _full((GD, HID)),
                  _full((1, HID)), _full((HID, HID)), _full((1, HID)),
                  _full((HID, 1)), _full((1, 1))],
        out_specs=[pl.BlockSpec((BN, HID), lambda i: (i, 0)),
                   pl.BlockSpec((BN, 1), lambda i: (i, 0)),
                   pl.BlockSpec((G, 1), lambda i: (0, 0))],
        out_shape=[jax.ShapeDtypeStruct((NP, HID), f32),
                   jax.ShapeDtypeStruct((NP, 1), f32),
                   jax.ShapeDtypeStruct((G, 1), f32)],
    )
    h5, attn, normg = tail1(
        h, acc2.reshape(2, NP, HID), conv_bias[LAYERS - 1].reshape(1, HID),
        bidx3, global_features, ga_W0[:HID], ga_W0[HID:],
        ga_b0.reshape(1, HID), ga_W1, ga_b1.reshape(1, HID), ga_W2,
        ga_b2.reshape(1, 1))

    tail2 = pl.pallas_call(
        _tail2_body,
        grid=(NP // BN,),
        in_specs=[pl.BlockSpec((BN, HID), lambda i: (i, 0)),
                  pl.BlockSpec((BN, 1), lambda i: (i, 0)),
                  _full((G, 1)),
                  pl.BlockSpec((1, 1, BN), lambda i: (i, 0, 0)),
                  _full((HID, HID)), _full((1, HID)),
                  _full((HID, 1)), _full((1, 1))],
        out_specs=[pl.BlockSpec((G, HID), lambda i: (0, 0)),
                   pl.BlockSpec((G, 1), lambda i: (0, 0))],
        out_shape=[jax.ShapeDtypeStruct((G, HID), f32),
                   jax.ShapeDtypeStruct((G, 1), f32)],
    )
    _, o = tail2(h5, attn, normg, bidx3, out_W0, out_b0.reshape(1, HID),
                 out_W1, out_b1.reshape(1, 1))
    return o.reshape(G)
